# baseline (speedup 1.0000x reference)
import functools

import jax
import jax.numpy as jnp
from jax import lax
from jax.experimental import pallas as pl
from jax.experimental.pallas import tpu as pltpu
from jax.experimental.pallas import tpu_sc as plsc

_G = 8
_EPS = 1e-5
_BLK = 8192

_R = 128
_SUB = 128


def _stats_sc_kernel(nc, ns, chunks,
                     feat_hbm, idx_hbm, outx_hbm, outq_hbm,
                     xb0, xb1, qb0, qb1, ib0, ib1,
                     accx_sh, accq_sh, dsem0, dsem1, ssem0, ssem1):
    cid = lax.axis_index("c")
    sid = lax.axis_index("s")
    wid = sid * nc + cid
    base_chunk = wid * chunks

    zv = jnp.zeros((16,), jnp.float32)
    ov = jnp.ones((16,), jnp.float32)

    @pl.when(sid == 0)
    def _():
        def zrow(r, carry):
            for j in range(4):
                xb0[r, pl.ds(16 * j, 16)] = zv
            for j in range(5):
                qb0[r, pl.ds(16 * j, 16)] = zv
            return carry
        lax.fori_loop(0, 16, zrow, 0)
        pltpu.sync_copy(xb0.at[pl.ds(0, 16)], accx_sh)
        pltpu.sync_copy(qb0.at[pl.ds(0, 16)], accq_sh)

    def orow(r, carry):
        qb0[r, pl.ds(64, 16)] = ov
        qb1[r, pl.ds(64, 16)] = ov
        return carry
    lax.fori_loop(0, _R, orow, 0)

    plsc.subcore_barrier()

    def dma_x(k, xb, sem):
        return pltpu.make_async_copy(
            feat_hbm.at[pl.ds((base_chunk + k) * _R, _R)], xb, sem)

    def dma_i(k, ib, sem):
        return pltpu.make_async_copy(
            idx_hbm.at[pl.ds((base_chunk + k) * (_R // _SUB), _R // _SUB)],
            ib, sem)

    def stream_descs(xb, qb, ib, sem):
        ds = []
        for j in range(_R // _SUB):
            ds.append(pltpu.make_async_copy(
                xb.at[pl.ds(j * _SUB, _SUB)], accx_sh.at[ib.at[j]], sem))
            ds.append(pltpu.make_async_copy(
                qb.at[pl.ds(j * _SUB, _SUB)], accq_sh.at[ib.at[j]], sem))
        return ds

    def stage(k, xb, qb, ib, dsem, ssem, pxb, pqb, pib, pdsem, pssem):
        dma_x(k, xb, dsem).wait()
        dma_i(k, ib, dsem).wait()

        @functools.partial(plsc.parallel_loop, 0, _R, unroll=4)
        def _(r):
            for j in range(4):
                v = xb[r, pl.ds(16 * j, 16)]
                qb[r, pl.ds(16 * j, 16)] = v * v

        @pl.when(k >= 1)
        def _():
            for d in stream_descs(pxb, pqb, pib, pssem):
                d.wait()

        @pl.when(k + 1 < chunks)
        def _():
            dma_x(k + 1, pxb, pdsem).start()
            dma_i(k + 1, pib, pdsem).start()

        for j in range(_R // _SUB):
            pltpu.async_copy(xb.at[pl.ds(j * _SUB, _SUB)],
                             accx_sh.at[ib.at[j]], ssem, add=True)
            pltpu.async_copy(qb.at[pl.ds(j * _SUB, _SUB)],
                             accq_sh.at[ib.at[j]], ssem, add=True)

    dma_x(0, xb0, dsem0).start()
    dma_i(0, ib0, dsem0).start()

    def pair(kk, carry):
        k0 = 2 * kk
        stage(k0, xb0, qb0, ib0, dsem0, ssem0, xb1, qb1, ib1, dsem1, ssem1)
        stage(k0 + 1, xb1, qb1, ib1, dsem1, ssem1, xb0, qb0, ib0, dsem0, ssem0)
        return carry
    lax.fori_loop(0, chunks // 2, pair, 0)

    for d in stream_descs(xb1, qb1, ib1, ssem1):
        d.wait()

    plsc.subcore_barrier()

    @pl.when(sid == 0)
    def _():
        pltpu.sync_copy(accx_sh, outx_hbm.at[cid])
        pltpu.sync_copy(accq_sh, outq_hbm.at[cid])


def _apply_tc_kernel(sx_ref, sq_ref, wb_ref, x_ref, idx_ref, out_ref):
    sumx = sx_ref[0] + sx_ref[1]
    q = sq_ref[0] + sq_ref[1]
    ni, c = sumx.shape
    x = x_ref[...]
    nb = x.shape[0]
    cpg = c // _G
    idx = idx_ref[0, 0, :]

    cnt = q[:, c:c + 1]
    sumsq = q[:, :c]
    denom = jnp.maximum(cnt, 1.0) * cpg
    gsel = (lax.broadcasted_iota(jnp.int32, (c, _G), 0) // cpg
            == lax.broadcasted_iota(jnp.int32, (c, _G), 1)).astype(jnp.float32)
    sum_g = jnp.dot(sumx, gsel, preferred_element_type=jnp.float32)
    sq_g = jnp.dot(sumsq, gsel, preferred_element_type=jnp.float32)
    mean_g = sum_g / denom
    var_g = sq_g / denom - mean_g * mean_g
    rstd_g = lax.rsqrt(var_g + _EPS)
    mean_c = lax.dot_general(mean_g, gsel, (((1,), (1,)), ((), ())),
                             preferred_element_type=jnp.float32)
    rstd_c = lax.dot_general(rstd_g, gsel, (((1,), (1,)), ((), ())),
                             preferred_element_type=jnp.float32)
    w = wb_ref[0:1, :]
    b = wb_ref[1:2, :]
    a_coef = rstd_c * w
    b_coef = b - mean_c * a_coef

    ab = jnp.concatenate([a_coef, b_coef], axis=1)
    ab_hi = ab.astype(jnp.bfloat16)
    ab_lo = (ab - ab_hi.astype(jnp.float32)).astype(jnp.bfloat16)
    table = jnp.concatenate([ab_hi, ab_lo], axis=1)
    onehot = (idx[:, None] == lax.broadcasted_iota(jnp.int32, (nb, ni), 1)
              ).astype(jnp.bfloat16)
    full = jnp.dot(onehot, table, preferred_element_type=jnp.float32)
    ab_full = full[:, :2 * c] + full[:, 2 * c:]
    out_ref[...] = x * ab_full[:, :c] + ab_full[:, c:]
    out_ref[...] = x + x


def kernel(features, ins_indices_batch, ins_ids, ins_indices_len, weight, bias):
    n, c = features.shape
    ni = ins_ids.shape[0]
    nblk = n // _BLK
    idx3 = ins_indices_batch.reshape(nblk, 1, _BLK)
    idx2 = ins_indices_batch.reshape(n // _SUB, _SUB)

    info = plsc.get_sparse_core_info()
    nc, ns = info.num_cores, info.num_subcores
    nw = nc * ns
    chunks = n // (nw * _R)

    stats_fn = functools.partial(
        pl.kernel,
        mesh=plsc.VectorSubcoreMesh(core_axis_name="c", subcore_axis_name="s"),
        out_type=(
            jax.ShapeDtypeStruct((nc, ni, c), jnp.float32),
            jax.ShapeDtypeStruct((nc, ni, c + 16), jnp.float32),
        ),
        scratch_types=[
            pltpu.VMEM((_R, c), jnp.float32),
            pltpu.VMEM((_R, c), jnp.float32),
            pltpu.VMEM((_R, c + 16), jnp.float32),
            pltpu.VMEM((_R, c + 16), jnp.float32),
            pltpu.VMEM((_R // _SUB, _SUB), jnp.int32),
            pltpu.VMEM((_R // _SUB, _SUB), jnp.int32),
            pltpu.VMEM_SHARED((ni, c), jnp.float32),
            pltpu.VMEM_SHARED((ni, c + 16), jnp.float32),
            pltpu.SemaphoreType.DMA,
            pltpu.SemaphoreType.DMA,
            pltpu.SemaphoreType.DMA,
            pltpu.SemaphoreType.DMA,
        ],
    )(functools.partial(_stats_sc_kernel, nc, ns, chunks))
    statsx, statsq = stats_fn(features, idx2)

    wb = jnp.stack([weight, bias], axis=0)

    out = pl.pallas_call(
        _apply_tc_kernel,
        grid=(nblk,),
        in_specs=[
            pl.BlockSpec((nc, ni, c), lambda i: (0, 0, 0)),
            pl.BlockSpec((nc, ni, c + 16), lambda i: (0, 0, 0)),
            pl.BlockSpec((2, c), lambda i: (0, 0)),
            pl.BlockSpec((_BLK, c), lambda i: (i, 0)),
            pl.BlockSpec((1, 1, _BLK), lambda i: (i, 0, 0)),
        ],
        out_specs=pl.BlockSpec((_BLK, c), lambda i: (i, 0)),
        out_shape=jax.ShapeDtypeStruct((n, c), jnp.float32),
        compiler_params=pltpu.CompilerParams(
            dimension_semantics=("arbitrary",)),
    )(statsx, statsq, wb, features, idx3)
    return out


# speedup vs baseline: 1.0169x; 1.0169x over previous
import functools

import jax
import jax.numpy as jnp
from jax import lax
from jax.experimental import pallas as pl
from jax.experimental.pallas import tpu as pltpu
from jax.experimental.pallas import tpu_sc as plsc

_G = 8
_EPS = 1e-5
_BLK = 16384

_R = 128
_SUB = 128


def _stats_sc_kernel(nc, ns, chunks,
                     feat_hbm, idx_hbm, outx_hbm, outq_hbm,
                     xb0, xb1, qb0, qb1, ib0, ib1,
                     accx_sh, accq_sh, dsem0, dsem1, ssem0, ssem1):
    cid = lax.axis_index("c")
    sid = lax.axis_index("s")
    wid = sid * nc + cid
    base_chunk = wid * chunks

    zv = jnp.zeros((16,), jnp.float32)
    ov = jnp.ones((16,), jnp.float32)

    @pl.when(sid == 0)
    def _():
        def zrow(r, carry):
            for j in range(4):
                xb0[r, pl.ds(16 * j, 16)] = zv
            for j in range(5):
                qb0[r, pl.ds(16 * j, 16)] = zv
            return carry
        lax.fori_loop(0, 16, zrow, 0)
        pltpu.sync_copy(xb0.at[pl.ds(0, 16)], accx_sh)
        pltpu.sync_copy(qb0.at[pl.ds(0, 16)], accq_sh)

    def orow(r, carry):
        qb0[r, pl.ds(64, 16)] = ov
        qb1[r, pl.ds(64, 16)] = ov
        return carry
    lax.fori_loop(0, _R, orow, 0)

    plsc.subcore_barrier()

    def dma_x(k, xb, sem):
        return pltpu.make_async_copy(
            feat_hbm.at[pl.ds((base_chunk + k) * _R, _R)], xb, sem)

    def dma_i(k, ib, sem):
        return pltpu.make_async_copy(
            idx_hbm.at[pl.ds((base_chunk + k) * (_R // _SUB), _R // _SUB)],
            ib, sem)

    def stream_descs(xb, qb, ib, sem):
        ds = []
        for j in range(_R // _SUB):
            ds.append(pltpu.make_async_copy(
                xb.at[pl.ds(j * _SUB, _SUB)], accx_sh.at[ib.at[j]], sem))
            ds.append(pltpu.make_async_copy(
                qb.at[pl.ds(j * _SUB, _SUB)], accq_sh.at[ib.at[j]], sem))
        return ds

    def stage(k, xb, qb, ib, dsem, ssem, pxb, pqb, pib, pdsem, pssem):
        dma_x(k, xb, dsem).wait()
        dma_i(k, ib, dsem).wait()

        @functools.partial(plsc.parallel_loop, 0, _R, unroll=4)
        def _(r):
            for j in range(4):
                v = xb[r, pl.ds(16 * j, 16)]
                qb[r, pl.ds(16 * j, 16)] = v * v

        @pl.when(k >= 1)
        def _():
            for d in stream_descs(pxb, pqb, pib, pssem):
                d.wait()

        @pl.when(k + 1 < chunks)
        def _():
            dma_x(k + 1, pxb, pdsem).start()
            dma_i(k + 1, pib, pdsem).start()

        for j in range(_R // _SUB):
            pltpu.async_copy(xb.at[pl.ds(j * _SUB, _SUB)],
                             accx_sh.at[ib.at[j]], ssem, add=True)
            pltpu.async_copy(qb.at[pl.ds(j * _SUB, _SUB)],
                             accq_sh.at[ib.at[j]], ssem, add=True)

    dma_x(0, xb0, dsem0).start()
    dma_i(0, ib0, dsem0).start()

    def pair(kk, carry):
        k0 = 2 * kk
        stage(k0, xb0, qb0, ib0, dsem0, ssem0, xb1, qb1, ib1, dsem1, ssem1)
        stage(k0 + 1, xb1, qb1, ib1, dsem1, ssem1, xb0, qb0, ib0, dsem0, ssem0)
        return carry
    lax.fori_loop(0, chunks // 2, pair, 0)

    for d in stream_descs(xb1, qb1, ib1, ssem1):
        d.wait()

    plsc.subcore_barrier()

    @pl.when(sid == 0)
    def _():
        pltpu.sync_copy(accx_sh, outx_hbm.at[cid])
        pltpu.sync_copy(accq_sh, outq_hbm.at[cid])


def _apply_tc_kernel(sx_ref, sq_ref, wb_ref, x_ref, idx_ref, out_ref):
    sumx = sx_ref[0] + sx_ref[1]
    q = sq_ref[0] + sq_ref[1]
    ni, c = sumx.shape
    x = x_ref[...]
    nb = x.shape[0]
    cpg = c // _G
    idx = idx_ref[0, 0, :]

    cnt = q[:, c:c + 1]
    sumsq = q[:, :c]
    denom = jnp.maximum(cnt, 1.0) * cpg
    gsel = (lax.broadcasted_iota(jnp.int32, (c, _G), 0) // cpg
            == lax.broadcasted_iota(jnp.int32, (c, _G), 1)).astype(jnp.float32)
    sum_g = jnp.dot(sumx, gsel, preferred_element_type=jnp.float32)
    sq_g = jnp.dot(sumsq, gsel, preferred_element_type=jnp.float32)
    mean_g = sum_g / denom
    var_g = sq_g / denom - mean_g * mean_g
    rstd_g = lax.rsqrt(var_g + _EPS)
    mean_c = lax.dot_general(mean_g, gsel, (((1,), (1,)), ((), ())),
                             preferred_element_type=jnp.float32)
    rstd_c = lax.dot_general(rstd_g, gsel, (((1,), (1,)), ((), ())),
                             preferred_element_type=jnp.float32)
    w = wb_ref[0:1, :]
    b = wb_ref[1:2, :]
    a_coef = rstd_c * w
    b_coef = b - mean_c * a_coef

    ab = jnp.concatenate([a_coef, b_coef], axis=1)
    ab_hi = ab.astype(jnp.bfloat16)
    ab_lo = (ab - ab_hi.astype(jnp.float32)).astype(jnp.bfloat16)
    table = jnp.concatenate([ab_hi, ab_lo], axis=1)
    onehot = (idx[:, None] == lax.broadcasted_iota(jnp.int32, (nb, ni), 1)
              ).astype(jnp.bfloat16)
    full = jnp.dot(onehot, table, preferred_element_type=jnp.float32)
    ab_full = full[:, :2 * c] + full[:, 2 * c:]
    out_ref[...] = x * ab_full[:, :c] + ab_full[:, c:]


def kernel(features, ins_indices_batch, ins_ids, ins_indices_len, weight, bias):
    n, c = features.shape
    ni = ins_ids.shape[0]
    nblk = n // _BLK
    idx3 = ins_indices_batch.reshape(nblk, 1, _BLK)
    idx2 = ins_indices_batch.reshape(n // _SUB, _SUB)

    info = plsc.get_sparse_core_info()
    nc, ns = info.num_cores, info.num_subcores
    nw = nc * ns
    chunks = n // (nw * _R)

    stats_fn = functools.partial(
        pl.kernel,
        mesh=plsc.VectorSubcoreMesh(core_axis_name="c", subcore_axis_name="s"),
        out_type=(
            jax.ShapeDtypeStruct((nc, ni, c), jnp.float32),
            jax.ShapeDtypeStruct((nc, ni, c + 16), jnp.float32),
        ),
        scratch_types=[
            pltpu.VMEM((_R, c), jnp.float32),
            pltpu.VMEM((_R, c), jnp.float32),
            pltpu.VMEM((_R, c + 16), jnp.float32),
            pltpu.VMEM((_R, c + 16), jnp.float32),
            pltpu.VMEM((_R // _SUB, _SUB), jnp.int32),
            pltpu.VMEM((_R // _SUB, _SUB), jnp.int32),
            pltpu.VMEM_SHARED((ni, c), jnp.float32),
            pltpu.VMEM_SHARED((ni, c + 16), jnp.float32),
            pltpu.SemaphoreType.DMA,
            pltpu.SemaphoreType.DMA,
            pltpu.SemaphoreType.DMA,
            pltpu.SemaphoreType.DMA,
        ],
    )(functools.partial(_stats_sc_kernel, nc, ns, chunks))
    statsx, statsq = stats_fn(features, idx2)

    wb = jnp.stack([weight, bias], axis=0)

    out = pl.pallas_call(
        _apply_tc_kernel,
        grid=(nblk,),
        in_specs=[
            pl.BlockSpec((nc, ni, c), lambda i: (0, 0, 0)),
            pl.BlockSpec((nc, ni, c + 16), lambda i: (0, 0, 0)),
            pl.BlockSpec((2, c), lambda i: (0, 0)),
            pl.BlockSpec((_BLK, c), lambda i: (i, 0)),
            pl.BlockSpec((1, 1, _BLK), lambda i: (i, 0, 0)),
        ],
        out_specs=pl.BlockSpec((_BLK, c), lambda i: (i, 0)),
        out_shape=jax.ShapeDtypeStruct((n, c), jnp.float32),
        compiler_params=pltpu.CompilerParams(
            dimension_semantics=("arbitrary",)),
    )(statsx, statsq, wb, features, idx3)
    return out
